# Initial kernel scaffold; baseline (speedup 1.0000x reference)
import jax
import jax.numpy as jnp
from jax.experimental import pallas as pl


def kernel(x, edge_attr, W1_rel, b1_rel, W1_root, b1_root, W2_rel, b2_rel, W2_root, b2_root, W3_rel, b3_rel, W3_root, b3_root, W_lin, b_lin, edge_index, batch):
    raise NotImplementedError("write your pallas kernel here")


# speedup vs baseline: 5.7868x; 5.7868x over previous
import functools

import jax
import jax.numpy as jnp
from jax import lax
from jax.experimental import pallas as pl
from jax.experimental.pallas import tpu as pltpu
from jax.experimental.pallas import tpu_sc as plsc

N = 10000
E = 320000
D = 128
H = 32
C = 2
G = 64

NC = 2
NS = 16
NW = NC * NS
EP = E // NW
K = 80
NCHUNK = EP // K
SLAB = 624
REM = N - NS * SLAB


def _sc_segment_sum(hp, src, dst, zeros):
  mesh = plsc.VectorSubcoreMesh(core_axis_name="c", subcore_axis_name="s")

  @functools.partial(
      pl.kernel,
      out_type=jax.ShapeDtypeStruct((NC * N, H), jnp.float32),
      mesh=mesh,
      scratch_types=[
          pltpu.VMEM((K,), jnp.int32),
          pltpu.VMEM((K,), jnp.int32),
          pltpu.VMEM((K, H), jnp.float32),
          pltpu.VMEM_SHARED((N, H), jnp.float32),
          pltpu.SemaphoreType.DMA,
      ],
      compiler_params=pltpu.CompilerParams(use_tc_tiling_on_sc=False),
  )
  def seg_kernel(hp_hbm, src_hbm, dst_hbm, z_hbm, out_hbm,
                 src_v, dst_v, rows_v, acc_sh, sem):
    cid = lax.axis_index("c")
    sid = lax.axis_index("s")

    pltpu.sync_copy(z_hbm.at[pl.ds(0, SLAB)],
                    acc_sh.at[pl.ds(sid * SLAB, SLAB)])

    @pl.when(sid == NS - 1)
    def _():
      pltpu.sync_copy(z_hbm.at[pl.ds(0, REM)],
                      acc_sh.at[pl.ds(NS * SLAB, REM)])

    plsc.subcore_barrier()

    base = (cid * NS + sid) * EP

    def chunk_body(i, carry):
      off = base + i * K
      pltpu.sync_copy(src_hbm.at[pl.ds(off, K)], src_v)
      pltpu.sync_copy(dst_hbm.at[pl.ds(off, K)], dst_v)
      pltpu.async_copy(hp_hbm.at[src_v], rows_v, sem).wait()
      pltpu.sync_copy(rows_v, acc_sh.at[dst_v], add=True)
      return carry

    lax.fori_loop(0, NCHUNK, chunk_body, 0)
    plsc.subcore_barrier()

    pltpu.sync_copy(acc_sh.at[pl.ds(sid * SLAB, SLAB)],
                    out_hbm.at[pl.ds(cid * N + sid * SLAB, SLAB)])

    @pl.when(sid == NS - 1)
    def _():
      pltpu.sync_copy(acc_sh.at[pl.ds(NS * SLAB, REM)],
                      out_hbm.at[pl.ds(cid * N + NS * SLAB, REM)])

  return seg_kernel(hp, src, dst, zeros)


def _elu(x):
  return jnp.where(x > 0, x, jnp.exp(jnp.minimum(x, 0.0)) - 1.0)


_R1 = 2000


def _tc_pre_body(x_ref, wr_ref, wo_ref, b_ref, hp_ref, root_ref):
  xb = x_ref[...]
  hp_ref[...] = jnp.dot(xb, wr_ref[...], preferred_element_type=jnp.float32)
  root_ref[...] = (
      jnp.dot(xb, wo_ref[...], preferred_element_type=jnp.float32) + b_ref[...])


def _tc_pre(x, Wr, Wo, b):
  grid = (N // _R1,)
  return pl.pallas_call(
      _tc_pre_body,
      grid=grid,
      in_specs=[
          pl.BlockSpec((_R1, D), lambda i: (i, 0)),
          pl.BlockSpec((D, H), lambda i: (0, 0)),
          pl.BlockSpec((D, H), lambda i: (0, 0)),
          pl.BlockSpec((1, H), lambda i: (0, 0)),
      ],
      out_specs=[
          pl.BlockSpec((_R1, H), lambda i: (i, 0)),
          pl.BlockSpec((_R1, H), lambda i: (i, 0)),
      ],
      out_shape=[
          jax.ShapeDtypeStruct((N, H), jnp.float32),
          jax.ShapeDtypeStruct((N, H), jnp.float32),
      ],
  )(x, Wr, Wo, b)


def _tc_mid_body(p_ref, rprev_ref, wr_ref, wo_ref, b_ref, hp_ref, root_ref):
  p = p_ref[...]
  h = _elu(p[0] + p[1] + rprev_ref[...])
  hp_ref[...] = jnp.dot(h, wr_ref[...], preferred_element_type=jnp.float32)
  root_ref[...] = (
      jnp.dot(h, wo_ref[...], preferred_element_type=jnp.float32) + b_ref[...])


def _tc_mid(partials, root_prev, Wr, Wo, b):
  p3 = partials.reshape(NC, N, H)
  grid = (N // _R1,)
  return pl.pallas_call(
      _tc_mid_body,
      grid=grid,
      in_specs=[
          pl.BlockSpec((NC, _R1, H), lambda i: (0, i, 0)),
          pl.BlockSpec((_R1, H), lambda i: (i, 0)),
          pl.BlockSpec((H, H), lambda i: (0, 0)),
          pl.BlockSpec((H, H), lambda i: (0, 0)),
          pl.BlockSpec((1, H), lambda i: (0, 0)),
      ],
      out_specs=[
          pl.BlockSpec((_R1, H), lambda i: (i, 0)),
          pl.BlockSpec((_R1, H), lambda i: (i, 0)),
      ],
      out_shape=[
          jax.ShapeDtypeStruct((N, H), jnp.float32),
          jax.ShapeDtypeStruct((N, H), jnp.float32),
      ],
  )(p3, root_prev, Wr, Wo, b)


_R2 = 2000


def _tc_pool_body(p_ref, root_ref, batch_ref, sums_ref, counts_ref):
  i = pl.program_id(0)
  p = p_ref[...]
  h = _elu(p[0] + p[1] + root_ref[...])
  gid = lax.broadcasted_iota(jnp.int32, (G, _R2), 0)
  mask = (gid == batch_ref[0]).astype(jnp.float32)

  @pl.when(i == 0)
  def _():
    sums_ref[...] = jnp.zeros_like(sums_ref)
    counts_ref[...] = jnp.zeros_like(counts_ref)

  sums_ref[...] += jnp.dot(mask, h, preferred_element_type=jnp.float32)
  counts_ref[...] += jnp.sum(mask, axis=1, keepdims=True)


def _tc_pool(partials, root, batch3d):
  p3 = partials.reshape(NC, N, H)
  grid = (N // _R2,)
  return pl.pallas_call(
      _tc_pool_body,
      grid=grid,
      in_specs=[
          pl.BlockSpec((NC, _R2, H), lambda i: (0, i, 0)),
          pl.BlockSpec((_R2, H), lambda i: (i, 0)),
          pl.BlockSpec((1, 1, _R2), lambda i: (i, 0, 0)),
      ],
      out_specs=[
          pl.BlockSpec((G, H), lambda i: (0, 0)),
          pl.BlockSpec((G, 1), lambda i: (0, 0)),
      ],
      out_shape=[
          jax.ShapeDtypeStruct((G, H), jnp.float32),
          jax.ShapeDtypeStruct((G, 1), jnp.float32),
      ],
  )(p3, root, batch3d)


def _tc_head_body(sums_ref, counts_ref, wl_ref, bl_ref, out_ref):
  pooled = sums_ref[...] / jnp.maximum(counts_ref[...], 1.0)
  logits = (
      jnp.dot(pooled, wl_ref[...], preferred_element_type=jnp.float32)
      + bl_ref[...])
  m = jnp.max(logits, axis=1, keepdims=True)
  lse = m + jnp.log(jnp.sum(jnp.exp(logits - m), axis=1, keepdims=True))
  out_ref[...] = logits - lse


def _tc_head(sums, counts, Wl, bl):
  return pl.pallas_call(
      _tc_head_body,
      out_shape=jax.ShapeDtypeStruct((G, C), jnp.float32),
  )(sums, counts, Wl, bl)


def kernel(x, edge_attr, W1_rel, b1_rel, W1_root, b1_root, W2_rel, b2_rel,
           W2_root, b2_root, W3_rel, b3_rel, W3_root, b3_root, W_lin, b_lin,
           edge_index, batch):
  del edge_attr
  src = edge_index[0]
  dst = edge_index[1]
  zeros = jnp.zeros((SLAB, H), jnp.float32)
  batch3d = batch.reshape(N // _R2, 1, _R2)

  b1 = (b1_rel + b1_root).reshape(1, H)
  b2 = (b2_rel + b2_root).reshape(1, H)
  b3 = (b3_rel + b3_root).reshape(1, H)

  hp1, root1 = _tc_pre(x, W1_rel, W1_root, b1)
  p1 = _sc_segment_sum(hp1, src, dst, zeros)
  hp2, root2 = _tc_mid(p1, root1, W2_rel, W2_root, b2)
  p2 = _sc_segment_sum(hp2, src, dst, zeros)
  hp3, root3 = _tc_mid(p2, root2, W3_rel, W3_root, b3)
  p3 = _sc_segment_sum(hp3, src, dst, zeros)
  sums, counts = _tc_pool(p3, root3, batch3d)
  return _tc_head(sums, counts, W_lin, b_lin.reshape(1, C))
